# baseline (device time: 180296 ns/iter reference)
import os

import jax
import jax.numpy as jnp
from jax import lax
from jax.experimental import pallas as pl
from jax.experimental.pallas import tpu as pltpu

N_DEV = 4
NBLK = 256
S16 = 8.0 / 32767.0


def kernel(x, w_mat):
    M, K = x.shape
    _, N = w_mat.shape
    NB = N // N_DEV
    NSTEPS = N // NBLK
    GPB = NB // NBLK

    def body(x_ref, w_ref, out_ref, y16, qsend, qrecv, amax_buf, stg,
             wsems, dsend_sems, drecv_sems, asend_sems, arecv_sems):
        n = pl.program_id(0)
        my = lax.axis_index("i")

        part = jnp.maximum(
            jnp.dot(x_ref[...], w_ref[...],
                    preferred_element_type=jnp.float32),
            0.0,
        )
        y16[pl.ds(n, 1)] = jnp.clip(
            jnp.round(part / S16), 0.0, 32767.0).astype(jnp.int16)[None]

        m = jnp.full((8, 128), jnp.max(part), jnp.float32)

        @pl.when(n == 0)
        def _():
            amax_buf[0] = m

        @pl.when(n != 0)
        def _():
            amax_buf[0] = jnp.maximum(amax_buf[0], m)

        @pl.when(n == NSTEPS - 1)
        def _():
            amax_rdmas = []
            for d in range(1, N_DEV):
                tgt = lax.rem(my + d, N_DEV)
                r = pltpu.make_async_remote_copy(
                    src_ref=amax_buf.at[0],
                    dst_ref=amax_buf.at[d],
                    send_sem=asend_sems.at[d],
                    recv_sem=arecv_sems.at[d],
                    device_id=(tgt,),
                    device_id_type=pl.DeviceIdType.MESH,
                )
                r.start()
                amax_rdmas.append(r)
            for r in amax_rdmas:
                r.wait_recv()

            amax = jnp.max(amax_buf[...])
            scale = amax / 127.0

            data_rdmas = {}
            for d in range(1, N_DEV):
                tgt = lax.rem(my + d, N_DEV)
                for g in range(GPB):
                    s = tgt * GPB + g
                    y = y16[pl.ds(s, 1)][0].astype(jnp.float32) * S16
                    q = jnp.clip(jnp.round(y / scale), -127.0, 127.0)
                    qsend[d - 1, g] = q.astype(jnp.int8)
                    r = pltpu.make_async_remote_copy(
                        src_ref=qsend.at[d - 1, g],
                        dst_ref=qrecv.at[d - 1, g],
                        send_sem=dsend_sems.at[d, g],
                        recv_sem=drecv_sems.at[d, g],
                        device_id=(tgt,),
                        device_id_type=pl.DeviceIdType.MESH,
                    )
                    r.start()
                    data_rdmas[(d, g)] = r

            RC = 4
            MRC = M // RC
            prev_writes = []

            def assemble(g, v):
                for rc in range(RC):
                    stg[rc, :, g * NBLK:(g + 1) * NBLK] = (
                        v[rc * MRC:(rc + 1) * MRC, :])

            def flush_block(row0, d):
                cps = []
                for rc in range(RC):
                    cp = pltpu.make_async_copy(
                        stg.at[rc],
                        out_ref.at[pl.ds(row0 + rc * MRC, MRC), :],
                        wsems.at[d, rc])
                    cp.start()
                    cps.append(cp)
                return cps

            for g in range(GPB):
                s = my * GPB + g
                y = y16[pl.ds(s, 1)][0].astype(jnp.float32) * S16
                q = jnp.clip(jnp.round(y / scale), -127.0, 127.0)
                assemble(g, q * scale)
            prev_writes = flush_block(my * M, 0)

            for d in range(1, N_DEV):
                src = lax.rem(my - d + N_DEV, N_DEV)
                for g in range(GPB):
                    data_rdmas[(d, g)].wait_recv()
                    v = qrecv[d - 1, g].astype(jnp.float32) * scale
                    if g == 0:
                        for cp in prev_writes:
                            cp.wait()
                    assemble(g, v)
                prev_writes = flush_block(src * M, d)

            for cp in prev_writes:
                cp.wait()
            for r in amax_rdmas:
                r.wait_send()
            for r in data_rdmas.values():
                r.wait_send()

    return pl.pallas_call(
        body,
        grid=(NSTEPS,),
        out_shape=jax.ShapeDtypeStruct((N_DEV * M, NB), jnp.float32),
        in_specs=[
            pl.BlockSpec((M, K), lambda n: (0, 0)),
            pl.BlockSpec((K, NBLK), lambda n: (0, n)),
        ],
        out_specs=pl.BlockSpec(memory_space=pl.ANY),
        scratch_shapes=[
            pltpu.VMEM((NSTEPS, M, NBLK), jnp.int16),
            pltpu.VMEM((N_DEV - 1, NB // NBLK, M, NBLK), jnp.int8),
            pltpu.VMEM((N_DEV - 1, NB // NBLK, M, NBLK), jnp.int8),
            pltpu.VMEM((N_DEV, 8, 128), jnp.float32),
            pltpu.VMEM((4, M // 4, NB), jnp.float32),
            pltpu.SemaphoreType.DMA((N_DEV, NB // NBLK)),
            pltpu.SemaphoreType.DMA((N_DEV, NB // NBLK)),
            pltpu.SemaphoreType.DMA((N_DEV, NB // NBLK)),
            pltpu.SemaphoreType.DMA((N_DEV,)),
            pltpu.SemaphoreType.DMA((N_DEV,)),
        ],
        compiler_params=pltpu.CompilerParams(
            dimension_semantics=("arbitrary",),
            vmem_limit_bytes=128 * 1024 * 1024,
        ),
    )(x, w_mat)


# device time: 173614 ns/iter; 1.0385x vs baseline; 1.0385x over previous
import jax
import jax.numpy as jnp
from jax import lax
from jax.experimental import pallas as pl
from jax.experimental.pallas import tpu as pltpu

N_DEV = 4
NBLK = 1024
MCHUNK = 256


def _gemm(x, w_mat):
    M, K = x.shape
    _, N = w_mat.shape

    def body(x_ref, w_ref, y_ref, amax_ref):
        n = pl.program_id(0)
        part = jnp.maximum(
            jnp.dot(x_ref[...], w_ref[...],
                    preferred_element_type=jnp.float32),
            0.0,
        )
        y_ref[...] = part
        m = jnp.full((8, 128), jnp.max(part), jnp.float32)

        @pl.when(n == 0)
        def _():
            amax_ref[...] = m

        @pl.when(n != 0)
        def _():
            amax_ref[...] = jnp.maximum(amax_ref[...], m)

    blks = (N // NBLK) // N_DEV
    return pl.pallas_call(
        body,
        grid=(N // NBLK,),
        out_shape=(
            jax.ShapeDtypeStruct((N_DEV * M, N // N_DEV), jnp.float32),
            jax.ShapeDtypeStruct((8, 128), jnp.float32),
        ),
        in_specs=[
            pl.BlockSpec((M, K), lambda n: (0, 0)),
            pl.BlockSpec((K, NBLK), lambda n: (0, n)),
        ],
        out_specs=(
            pl.BlockSpec((M, NBLK), lambda n: (n // blks, n % blks)),
            pl.BlockSpec((8, 128), lambda n: (0, 0)),
        ),
        compiler_params=pltpu.CompilerParams(
            dimension_semantics=("arbitrary",),
            vmem_limit_bytes=128 * 1024 * 1024,
        ),
    )(x, w_mat)


def _a2a(y, amax_local):
    NB = y.shape[1]
    M = y.shape[0] // N_DEV

    def body(y_ref, amax_in, out_ref, bounce, qsend, qrecv, amax_buf,
             copy_sems, wsems, dsend_sems, drecv_sems, asend_sems,
             arecv_sems):
        my = lax.axis_index("i")

        amax_buf[0] = amax_in[...]
        amax_rdmas = []
        for d in range(1, N_DEV):
            tgt = lax.rem(my + d, N_DEV)
            r = pltpu.make_async_remote_copy(
                src_ref=amax_buf.at[0],
                dst_ref=amax_buf.at[d],
                send_sem=asend_sems.at[d],
                recv_sem=arecv_sems.at[d],
                device_id=(tgt,),
                device_id_type=pl.DeviceIdType.MESH,
            )
            r.start()
            amax_rdmas.append(r)

        NCH = M // MCHUNK

        reads = {}
        for d in [1, 2, 3, 0]:
            tgt = lax.rem(my + d, N_DEV)
            for c in range(NCH):
                rows = pl.ds(c * MCHUNK, MCHUNK)
                cp = pltpu.make_async_copy(
                    y_ref.at[pl.ds(tgt * M + c * MCHUNK, MCHUNK), :],
                    bounce.at[d, rows, :],
                    copy_sems.at[d, c],
                )
                cp.start()
                reads[(d, c)] = cp

        for r in amax_rdmas:
            r.wait_recv()
        amax = jnp.max(amax_buf[...])
        scale = amax / 127.0

        data_rdmas = {}
        for d in range(1, N_DEV):
            tgt = lax.rem(my + d, N_DEV)
            for c in range(NCH):
                rows = pl.ds(c * MCHUNK, MCHUNK)
                reads[(d, c)].wait()
                q = jnp.clip(jnp.round(bounce[d, rows, :] / scale),
                             -127.0, 127.0)
                qsend[d - 1, rows, :] = q.astype(jnp.int8)
                r = pltpu.make_async_remote_copy(
                    src_ref=qsend.at[d - 1, rows, :],
                    dst_ref=qrecv.at[d - 1, rows, :],
                    send_sem=dsend_sems.at[d, c],
                    recv_sem=drecv_sems.at[d, c],
                    device_id=(tgt,),
                    device_id_type=pl.DeviceIdType.MESH,
                )
                r.start()
                data_rdmas[(d, c)] = r

        writes = []
        for c in range(NCH):
            rows = pl.ds(c * MCHUNK, MCHUNK)
            reads[(0, c)].wait()
            q = jnp.clip(jnp.round(bounce[0, rows, :] / scale),
                         -127.0, 127.0)
            bounce[0, rows, :] = q * scale
            cp = pltpu.make_async_copy(
                bounce.at[0, rows, :],
                out_ref.at[pl.ds(my * M + c * MCHUNK, MCHUNK), :],
                wsems.at[0, c])
            cp.start()
            writes.append(cp)

        for d in range(1, N_DEV):
            src = lax.rem(my - d + N_DEV, N_DEV)
            for c in range(NCH):
                rows = pl.ds(c * MCHUNK, MCHUNK)
                data_rdmas[(d, c)].wait_recv()
                bounce[d, rows, :] = (
                    qrecv[d - 1, rows, :].astype(jnp.float32) * scale)
                cp = pltpu.make_async_copy(
                    bounce.at[d, rows, :],
                    out_ref.at[pl.ds(src * M + c * MCHUNK, MCHUNK), :],
                    wsems.at[d, c])
                cp.start()
                writes.append(cp)

        for cp in writes:
            cp.wait()
        for r in amax_rdmas:
            r.wait_send()
        for r in data_rdmas.values():
            r.wait_send()

    return pl.pallas_call(
        body,
        out_shape=jax.ShapeDtypeStruct((N_DEV * M, NB), jnp.float32),
        in_specs=[
            pl.BlockSpec(memory_space=pl.ANY),
            pl.BlockSpec((8, 128), memory_space=pltpu.VMEM),
        ],
        out_specs=pl.BlockSpec(memory_space=pl.ANY),
        scratch_shapes=[
            pltpu.VMEM((N_DEV, M, NB), jnp.float32),
            pltpu.VMEM((N_DEV - 1, M, NB), jnp.int8),
            pltpu.VMEM((N_DEV - 1, M, NB), jnp.int8),
            pltpu.VMEM((N_DEV, 8, 128), jnp.float32),
            pltpu.SemaphoreType.DMA((N_DEV, M // MCHUNK)),
            pltpu.SemaphoreType.DMA((N_DEV, M // MCHUNK)),
            pltpu.SemaphoreType.DMA((N_DEV, M // MCHUNK)),
            pltpu.SemaphoreType.DMA((N_DEV, M // MCHUNK)),
            pltpu.SemaphoreType.DMA((N_DEV,)),
            pltpu.SemaphoreType.DMA((N_DEV,)),
        ],
        input_output_aliases={0: 0},
        compiler_params=pltpu.CompilerParams(
            vmem_limit_bytes=128 * 1024 * 1024,
        ),
    )(y, amax_local)


def kernel(x, w_mat):
    y, amax_local = _gemm(x, w_mat)
    return _a2a(y, amax_local)


# device time: 171076 ns/iter; 1.0539x vs baseline; 1.0148x over previous
import jax
import jax.numpy as jnp
from jax import lax
from jax.experimental import pallas as pl
from jax.experimental.pallas import tpu as pltpu

N_DEV = 4
NBLK = 512
MCHUNK = 256


def _gemm(x, w_mat):
    M, K = x.shape
    _, N = w_mat.shape

    def body(x_ref, w_ref, y_ref, amax_ref):
        n = pl.program_id(0)
        part = jnp.maximum(
            jnp.dot(x_ref[...], w_ref[...],
                    preferred_element_type=jnp.float32),
            0.0,
        )
        y_ref[...] = part
        m = jnp.full((8, 128), jnp.max(part), jnp.float32)

        @pl.when(n == 0)
        def _():
            amax_ref[...] = m

        @pl.when(n != 0)
        def _():
            amax_ref[...] = jnp.maximum(amax_ref[...], m)

    blks = (N // NBLK) // N_DEV
    return pl.pallas_call(
        body,
        grid=(N // NBLK,),
        out_shape=(
            jax.ShapeDtypeStruct((N_DEV * M, N // N_DEV), jnp.float32),
            jax.ShapeDtypeStruct((8, 128), jnp.float32),
        ),
        in_specs=[
            pl.BlockSpec((M, K), lambda n: (0, 0)),
            pl.BlockSpec((K, NBLK), lambda n: (0, n)),
        ],
        out_specs=(
            pl.BlockSpec((M, NBLK), lambda n: (n // blks, n % blks)),
            pl.BlockSpec((8, 128), lambda n: (0, 0)),
        ),
        compiler_params=pltpu.CompilerParams(
            dimension_semantics=("arbitrary",),
            vmem_limit_bytes=128 * 1024 * 1024,
        ),
    )(x, w_mat)


def _a2a(y, amax_local):
    NB = y.shape[1]
    M = y.shape[0] // N_DEV

    def body(y_ref, amax_in, out_ref, bounce, qsend, qrecv, amax_buf,
             copy_sems, wsems, dsend_sems, drecv_sems, asend_sems,
             arecv_sems):
        my = lax.axis_index("i")

        amax_buf[0] = amax_in[...]
        amax_rdmas = []
        for d in range(1, N_DEV):
            tgt = lax.rem(my + d, N_DEV)
            r = pltpu.make_async_remote_copy(
                src_ref=amax_buf.at[0],
                dst_ref=amax_buf.at[d],
                send_sem=asend_sems.at[d],
                recv_sem=arecv_sems.at[d],
                device_id=(tgt,),
                device_id_type=pl.DeviceIdType.MESH,
            )
            r.start()
            amax_rdmas.append(r)

        NCH = M // MCHUNK

        reads = {}
        for d in [1, 2, 3, 0]:
            tgt = lax.rem(my + d, N_DEV)
            for c in range(NCH):
                rows = pl.ds(c * MCHUNK, MCHUNK)
                cp = pltpu.make_async_copy(
                    y_ref.at[pl.ds(tgt * M + c * MCHUNK, MCHUNK), :],
                    bounce.at[d, rows, :],
                    copy_sems.at[d, c],
                )
                cp.start()
                reads[(d, c)] = cp

        for r in amax_rdmas:
            r.wait_recv()
        amax = jnp.max(amax_buf[...])
        scale = amax / 127.0

        data_rdmas = {}
        for d in range(1, N_DEV):
            tgt = lax.rem(my + d, N_DEV)
            for c in range(NCH):
                rows = pl.ds(c * MCHUNK, MCHUNK)
                reads[(d, c)].wait()
                q = jnp.clip(jnp.round(bounce[d, rows, :] / scale),
                             -127.0, 127.0)
                qsend[d - 1, rows, :] = q.astype(jnp.int8)
                r = pltpu.make_async_remote_copy(
                    src_ref=qsend.at[d - 1, rows, :],
                    dst_ref=qrecv.at[d - 1, rows, :],
                    send_sem=dsend_sems.at[d, c],
                    recv_sem=drecv_sems.at[d, c],
                    device_id=(tgt,),
                    device_id_type=pl.DeviceIdType.MESH,
                )
                r.start()
                data_rdmas[(d, c)] = r

        writes = []
        for c in range(NCH):
            rows = pl.ds(c * MCHUNK, MCHUNK)
            reads[(0, c)].wait()
            q = jnp.clip(jnp.round(bounce[0, rows, :] / scale),
                         -127.0, 127.0)
            bounce[0, rows, :] = q * scale
            cp = pltpu.make_async_copy(
                bounce.at[0, rows, :],
                out_ref.at[pl.ds(my * M + c * MCHUNK, MCHUNK), :],
                wsems.at[0, c])
            cp.start()
            writes.append(cp)

        for d in range(1, N_DEV):
            src = lax.rem(my - d + N_DEV, N_DEV)
            for c in range(NCH):
                rows = pl.ds(c * MCHUNK, MCHUNK)
                data_rdmas[(d, c)].wait_recv()
                bounce[d, rows, :] = (
                    qrecv[d - 1, rows, :].astype(jnp.float32) * scale)
                cp = pltpu.make_async_copy(
                    bounce.at[d, rows, :],
                    out_ref.at[pl.ds(src * M + c * MCHUNK, MCHUNK), :],
                    wsems.at[d, c])
                cp.start()
                writes.append(cp)

        for cp in writes:
            cp.wait()
        for r in amax_rdmas:
            r.wait_send()
        for r in data_rdmas.values():
            r.wait_send()

    return pl.pallas_call(
        body,
        out_shape=jax.ShapeDtypeStruct((N_DEV * M, NB), jnp.float32),
        in_specs=[
            pl.BlockSpec(memory_space=pl.ANY),
            pl.BlockSpec((8, 128), memory_space=pltpu.VMEM),
        ],
        out_specs=pl.BlockSpec(memory_space=pl.ANY),
        scratch_shapes=[
            pltpu.VMEM((N_DEV, M, NB), jnp.float32),
            pltpu.VMEM((N_DEV - 1, M, NB), jnp.int8),
            pltpu.VMEM((N_DEV - 1, M, NB), jnp.int8),
            pltpu.VMEM((N_DEV, 8, 128), jnp.float32),
            pltpu.SemaphoreType.DMA((N_DEV, M // MCHUNK)),
            pltpu.SemaphoreType.DMA((N_DEV, M // MCHUNK)),
            pltpu.SemaphoreType.DMA((N_DEV, M // MCHUNK)),
            pltpu.SemaphoreType.DMA((N_DEV, M // MCHUNK)),
            pltpu.SemaphoreType.DMA((N_DEV,)),
            pltpu.SemaphoreType.DMA((N_DEV,)),
        ],
        input_output_aliases={0: 0},
        compiler_params=pltpu.CompilerParams(
            vmem_limit_bytes=128 * 1024 * 1024,
        ),
    )(y, amax_local)


def kernel(x, w_mat):
    y, amax_local = _gemm(x, w_mat)
    return _a2a(y, amax_local)
